# baseline (speedup 1.0000x reference)
import functools

import jax
import jax.numpy as jnp
from jax.experimental import pallas as pl
from jax.experimental.pallas import tpu as pltpu

_NUM_HEADS = 16
_SEQ = 2048
_HEAD_DIM = 64
_SCALE = 1.0 / (_HEAD_DIM ** 0.5)
_Q_CHUNK = 256


def _attn_kernel(q_ref, k_ref, v_ref, o_ref):
    outs = []
    for h in range(2):
        lo = h * _HEAD_DIM
        qh = (q_ref[0, :, lo:lo + _HEAD_DIM] * _SCALE).astype(jnp.bfloat16)
        kh = k_ref[0, :, lo:lo + _HEAD_DIM].astype(jnp.bfloat16)
        vh = v_ref[0, :, lo:lo + _HEAD_DIM].astype(jnp.bfloat16)
        o_chunks = []
        for c in range(_SEQ // _Q_CHUNK):
            qc = qh[c * _Q_CHUNK:(c + 1) * _Q_CHUNK, :]
            s = jax.lax.dot_general(
                qc, kh, (((1,), (1,)), ((), ())),
                preferred_element_type=jnp.float32
            ).astype(jnp.bfloat16)
            m = jnp.max(s, axis=1, keepdims=True)
            p = jnp.exp(s - m)
            l = jnp.sum(p.astype(jnp.float32), axis=1, keepdims=True)
            o = jax.lax.dot_general(
                p, vh, (((1,), (0,)), ((), ())),
                preferred_element_type=jnp.float32)
            o_chunks.append(o / l)
        outs.append(jnp.concatenate(o_chunks, axis=0))
    o_ref[0] = jnp.concatenate(outs, axis=1)


@jax.jit
def kernel(q, k, v):
    b, s, dm = q.shape
    grid = (_NUM_HEADS // 2,)
    spec = pl.BlockSpec((1, _SEQ, 2 * _HEAD_DIM), lambda h: (0, 0, h))
    out = pl.pallas_call(
        _attn_kernel,
        grid=grid,
        in_specs=[spec, spec, spec],
        out_specs=spec,
        out_shape=jax.ShapeDtypeStruct((b, s, dm), q.dtype),
    )(q, k, v)
    return out


# speedup vs baseline: 1.2258x; 1.2258x over previous
import functools

import jax
import jax.numpy as jnp
from jax.experimental import pallas as pl
from jax.experimental.pallas import tpu as pltpu

_NUM_HEADS = 16
_SEQ = 2048
_HEAD_DIM = 64
_SCALE = 1.0 / (_HEAD_DIM ** 0.5)
_Q_CHUNK = 512


def _attn_kernel(q_ref, k_ref, v_ref, o_ref):
    outs = []
    for h in range(2):
        lo = h * _HEAD_DIM
        qh = (q_ref[0, :, lo:lo + _HEAD_DIM] * _SCALE).astype(jnp.bfloat16)
        kh = k_ref[0, :, lo:lo + _HEAD_DIM].astype(jnp.bfloat16)
        vh = v_ref[0, :, lo:lo + _HEAD_DIM].astype(jnp.bfloat16)
        vaug = jnp.concatenate([vh, jnp.ones_like(vh)], axis=1)
        o_chunks = []
        for c in range(_SEQ // _Q_CHUNK):
            qc = qh[c * _Q_CHUNK:(c + 1) * _Q_CHUNK, :]
            s = jax.lax.dot_general(
                qc, kh, (((1,), (1,)), ((), ())),
                preferred_element_type=jnp.float32
            ).astype(jnp.bfloat16)
            m = jnp.max(s, axis=1, keepdims=True)
            p = jnp.exp(s - m)
            o2 = jax.lax.dot_general(
                p, vaug, (((1,), (0,)), ((), ())),
                preferred_element_type=jnp.float32)
            o_chunks.append(o2[:, :_HEAD_DIM] / o2[:, _HEAD_DIM:])
        outs.append(jnp.concatenate(o_chunks, axis=0))
    o_ref[0] = jnp.concatenate(outs, axis=1)


@jax.jit
def kernel(q, k, v):
    b, s, dm = q.shape
    grid = (_NUM_HEADS // 2,)
    spec = pl.BlockSpec((1, _SEQ, 2 * _HEAD_DIM), lambda h: (0, 0, h))
    out = pl.pallas_call(
        _attn_kernel,
        grid=grid,
        in_specs=[spec, spec, spec],
        out_specs=spec,
        out_shape=jax.ShapeDtypeStruct((b, s, dm), q.dtype),
    )(q, k, v)
    return out
